# baseline (speedup 1.0000x reference)
import jax
import jax.numpy as jnp
import numpy as np
from jax import lax
from jax.experimental import pallas as pl
from jax.experimental.pallas import tpu as pltpu
from jax.experimental.pallas import tpu_sc as plsc

_B = 16384
_C = 128
_K = 24
_DELTA_OMEGA = float(np.log(30.0) / 23.0)
_TAU_OFFSET = float(np.log(24.0))
_LN2 = float(np.log(2.0))

_P0 = -2.7941536765361863
_P1 = 5.069756316633883
_P2 = -3.5202188381464623
_P3 = 1.6101775468974928
_P4 = -0.40947558576670895
_P5 = 0.04392862784798757

_NC = 2
_NS = 16
_NW = _NC * _NS
_RPT = _B // _NW
_CHB = 128
_NCHB = _RPT // _CHB
_NG = _RPT // 16


def _log_poly(m):
    r = _P5
    r = r * m + _P4
    r = r * m + _P3
    r = r * m + _P2
    r = r * m + _P1
    return r * m + _P0


def _sc_body(tab, cellT, wf, bf, out,
             cbuf, albuf, ilo, ihi, dlo0, dhi0, dlo1, dhi1, wbuf, bbuf, obuf,
             sem0, sem1, sem2, sem3):
    wid = lax.axis_index("s") * _NC + lax.axis_index("c")
    base = wid * _RPT

    pltpu.sync_copy(cellT.at[:, pl.ds(base, _RPT)], cbuf)
    pltpu.sync_copy(wf, wbuf)
    pltpu.sync_copy(bf, bbuf)
    iota = lax.iota(jnp.int32, 16)

    wv = [[wbuf[pl.ds(o * _C + g * 16, 16)] for g in range(_C // 16)]
          for o in range(3)]
    zero = jnp.zeros((16,), jnp.float32)
    bvec = [jnp.where(iota == 0,
                      plsc.load_gather(bbuf, [jnp.full((16,), o, jnp.int32)]),
                      zero)
            for o in range(3)]

    def build(g, c):
        c0 = cbuf[0, pl.ds(g * 16, 16)]
        c1 = cbuf[1, pl.ds(g * 16, 16)]
        x = jnp.maximum(c0 * c1, 1e-20)
        bits = plsc.bitcast(x, jnp.int32)
        e = lax.shift_right_arithmetic(bits, 23) - 127
        mant = plsc.bitcast(
            jnp.bitwise_or(jnp.bitwise_and(bits, 0x7FFFFF), 0x3F800000),
            jnp.float32)
        lnx = (e.astype(jnp.float32) + _log_poly(mant)) * _LN2
        tau = (-0.5 * lnx - _TAU_OFFSET) * (1.0 / _DELTA_OMEGA)
        tcl = jnp.clip(tau, 0.0, float(_K - 2) + 0.999995)
        kl = tcl.astype(jnp.int32)
        al = jnp.clip(tau - kl.astype(jnp.float32), 0.0, 1.0)
        albuf[pl.ds(g * 16, 16)] = al
        idx = (base + g * 16 + iota) * _K + kl
        ilo[pl.ds(g * 16, 16)] = idx
        ihi[pl.ds(g * 16, 16)] = idx + 1
        return c

    lax.fori_loop(0, _NG, build, 0)

    bufs = [(dlo0, dhi0, sem0, sem1), (dlo1, dhi1, sem2, sem3)]

    def start(ch):
        dl, dh, sl, sh = bufs[ch % 2]
        pltpu.async_copy(tab.at[ilo.at[pl.ds(ch * _CHB, _CHB)]], dl, sl)
        pltpu.async_copy(tab.at[ihi.at[pl.ds(ch * _CHB, _CHB)]], dh, sh)

    def wait(ch):
        dl, dh, sl, sh = bufs[ch % 2]
        pltpu.make_async_copy(tab.at[ilo.at[pl.ds(0, _CHB)]], dl, sl).wait()
        pltpu.make_async_copy(tab.at[ihi.at[pl.ds(0, _CHB)]], dh, sh).wait()

    start(0)
    start(1)

    tau_init = (bvec[0], bvec[1], bvec[2])

    for ch in range(_NCHB):
        wait(ch)
        dl, dh, _, _ = bufs[ch % 2]

        def pair(j, ov):
            ov0, ov1, ov2 = ov
            sums = []
            for u in range(8):
                i = 8 * j + u
                a16 = plsc.load_gather(
                    albuf,
                    [jnp.full((16,), ch * _CHB, jnp.int32) + i])
                t0 = bvec[0]
                t1 = bvec[1]
                t2 = bvec[2]
                for g in range(_C // 16):
                    lo = dl[i, pl.ds(g * 16, 16)]
                    hi = dh[i, pl.ds(g * 16, 16)]
                    hr = lo + a16 * (hi - lo)
                    t0 = t0 + hr * wv[0][g]
                    t1 = t1 + hr * wv[1][g]
                    t2 = t2 + hr * wv[2][g]
                sums.append((jnp.sum(t0), jnp.sum(t1), jnp.sum(t2)))
            lane0 = jnp.bitwise_and(8 * j, 15)
            for u in range(8):
                su0, su1, su2 = sums[u]
                ov0 = jnp.where(iota == lane0 + u, su0, ov0)
                ov1 = jnp.where(iota == lane0 + u, su1, ov1)
                ov2 = jnp.where(iota == lane0 + u, su2, ov2)
            flush = lane0 == 8

            @pl.when(flush)
            def _():
                st = ch * _CHB + 8 * j - 8
                obuf[0, pl.ds(st, 16)] = ov0
                obuf[1, pl.ds(st, 16)] = ov1
                obuf[2, pl.ds(st, 16)] = ov2

            ov0 = jnp.where(flush, zero, ov0)
            ov1 = jnp.where(flush, zero, ov1)
            ov2 = jnp.where(flush, zero, ov2)
            return (ov0, ov1, ov2)

        lax.fori_loop(0, _CHB // 8, pair, tau_init)
        if ch + 2 < _NCHB:
            start(ch + 2)

    pltpu.sync_copy(obuf, out.at[:, pl.ds(base, _RPT)])


def _sc_call(tab, cellT, wf, bf):
    mesh = plsc.VectorSubcoreMesh(core_axis_name="c", subcore_axis_name="s")
    return pl.kernel(
        _sc_body,
        out_type=jax.ShapeDtypeStruct((3, _B), jnp.float32),
        mesh=mesh,
        compiler_params=pltpu.CompilerParams(use_tc_tiling_on_sc=False,
                                             needs_layout_passes=False),
        scratch_types=[
            pltpu.VMEM((2, _RPT), jnp.float32),
            pltpu.VMEM((_RPT,), jnp.float32),
            pltpu.VMEM((_RPT,), jnp.int32),
            pltpu.VMEM((_RPT,), jnp.int32),
            pltpu.VMEM((_CHB, _C), jnp.float32),
            pltpu.VMEM((_CHB, _C), jnp.float32),
            pltpu.VMEM((_CHB, _C), jnp.float32),
            pltpu.VMEM((_CHB, _C), jnp.float32),
            pltpu.VMEM((3 * _C,), jnp.float32),
            pltpu.VMEM((16,), jnp.float32),
            pltpu.VMEM((3, _RPT), jnp.float32),
            pltpu.SemaphoreType.DMA,
            pltpu.SemaphoreType.DMA,
            pltpu.SemaphoreType.DMA,
            pltpu.SemaphoreType.DMA,
        ],
    )(tab, cellT, wf, bf)


def kernel(h, cell, W, b):
    Bn, C_h, Kn = h.shape
    tab = jnp.transpose(h, (0, 2, 1)).reshape(Bn * Kn, C_h)
    cellT = cell.T
    wf = W.reshape(3 * C_h)
    bf = jnp.zeros((16,), jnp.float32).at[:3].set(b)
    o3 = _sc_call(tab, cellT, wf, bf)
    return o3.T


# speedup vs baseline: 1.0190x; 1.0190x over previous
import jax
import jax.numpy as jnp
import numpy as np
from jax import lax
from jax.experimental import pallas as pl
from jax.experimental.pallas import tpu as pltpu
from jax.experimental.pallas import tpu_sc as plsc

_B = 16384
_C = 128
_K = 24
_DELTA_OMEGA = float(np.log(30.0) / 23.0)
_TAU_OFFSET = float(np.log(24.0))
_LN2 = float(np.log(2.0))

_P0 = -2.7941536765361863
_P1 = 5.069756316633883
_P2 = -3.5202188381464623
_P3 = 1.6101775468974928
_P4 = -0.40947558576670895
_P5 = 0.04392862784798757

_NC = 2
_NS = 16
_NW = _NC * _NS
_RPT = _B // _NW
_CHB = 128
_NCHB = _RPT // _CHB
_NG = _RPT // 16


def _log_poly(m):
    r = _P5
    r = r * m + _P4
    r = r * m + _P3
    r = r * m + _P2
    r = r * m + _P1
    return r * m + _P0


def _sc_body(tab, cellT, wf, bf, out,
             cbuf, albuf, ilo, ihi, dlo0, dhi0, dlo1, dhi1, wbuf, bbuf, obuf,
             sem0, sem1, sem2, sem3):
    wid = lax.axis_index("s") * _NC + lax.axis_index("c")
    base = wid * _RPT

    pltpu.sync_copy(cellT.at[:, pl.ds(base, _RPT)], cbuf)
    pltpu.sync_copy(wf, wbuf)
    pltpu.sync_copy(bf, bbuf)
    iota = lax.iota(jnp.int32, 16)

    wv = [[wbuf[pl.ds(o * _C + g * 16, 16)] for g in range(_C // 16)]
          for o in range(3)]
    zero = jnp.zeros((16,), jnp.float32)
    bvec = [jnp.where(iota == 0,
                      plsc.load_gather(bbuf, [jnp.full((16,), o, jnp.int32)]),
                      zero)
            for o in range(3)]

    def build(g, c):
        c0 = cbuf[0, pl.ds(g * 16, 16)]
        c1 = cbuf[1, pl.ds(g * 16, 16)]
        x = jnp.maximum(c0 * c1, 1e-20)
        bits = plsc.bitcast(x, jnp.int32)
        e = lax.shift_right_arithmetic(bits, 23) - 127
        mant = plsc.bitcast(
            jnp.bitwise_or(jnp.bitwise_and(bits, 0x7FFFFF), 0x3F800000),
            jnp.float32)
        lnx = (e.astype(jnp.float32) + _log_poly(mant)) * _LN2
        tau = (-0.5 * lnx - _TAU_OFFSET) * (1.0 / _DELTA_OMEGA)
        tcl = jnp.clip(tau, 0.0, float(_K - 2) + 0.999995)
        kl = tcl.astype(jnp.int32)
        al = jnp.clip(tau - kl.astype(jnp.float32), 0.0, 1.0)
        albuf[pl.ds(g * 16, 16)] = al
        idx = (base + g * 16 + iota) * _K + kl
        ilo[pl.ds(g * 16, 16)] = idx
        ihi[pl.ds(g * 16, 16)] = idx + 1
        return c

    lax.fori_loop(0, _NG, build, 0)

    bufs = [(dlo0, dhi0, sem0, sem1), (dlo1, dhi1, sem2, sem3)]

    def start(ch):
        dl, dh, sl, sh = bufs[ch % 2]
        pltpu.async_copy(tab.at[ilo.at[pl.ds(ch * _CHB, _CHB)]], dl, sl)
        pltpu.async_copy(tab.at[ihi.at[pl.ds(ch * _CHB, _CHB)]], dh, sh)

    def wait(ch):
        dl, dh, sl, sh = bufs[ch % 2]
        pltpu.make_async_copy(tab.at[ilo.at[pl.ds(0, _CHB)]], dl, sl).wait()
        pltpu.make_async_copy(tab.at[ihi.at[pl.ds(0, _CHB)]], dh, sh).wait()

    start(0)
    start(1)

    tau_init = (bvec[0], bvec[1], bvec[2])

    for ch in range(_NCHB):
        wait(ch)
        dl, dh, _, _ = bufs[ch % 2]

        def pair(j, ov):
            ov0, ov1, ov2 = ov
            sums = []
            for u in range(4):
                i = 4 * j + u
                a16 = plsc.load_gather(
                    albuf,
                    [jnp.full((16,), ch * _CHB, jnp.int32) + i])
                t0 = bvec[0]
                t1 = bvec[1]
                t2 = bvec[2]
                for g in range(_C // 16):
                    lo = dl[i, pl.ds(g * 16, 16)]
                    hi = dh[i, pl.ds(g * 16, 16)]
                    hr = lo + a16 * (hi - lo)
                    t0 = t0 + hr * wv[0][g]
                    t1 = t1 + hr * wv[1][g]
                    t2 = t2 + hr * wv[2][g]
                sums.append((jnp.sum(t0), jnp.sum(t1), jnp.sum(t2)))
            lane0 = jnp.bitwise_and(4 * j, 15)
            for u in range(4):
                su0, su1, su2 = sums[u]
                ov0 = jnp.where(iota == lane0 + u, su0, ov0)
                ov1 = jnp.where(iota == lane0 + u, su1, ov1)
                ov2 = jnp.where(iota == lane0 + u, su2, ov2)
            flush = lane0 == 12

            @pl.when(flush)
            def _():
                st = ch * _CHB + 4 * j - 12
                obuf[0, pl.ds(st, 16)] = ov0
                obuf[1, pl.ds(st, 16)] = ov1
                obuf[2, pl.ds(st, 16)] = ov2

            ov0 = jnp.where(flush, zero, ov0)
            ov1 = jnp.where(flush, zero, ov1)
            ov2 = jnp.where(flush, zero, ov2)
            return (ov0, ov1, ov2)

        lax.fori_loop(0, _CHB // 4, pair, tau_init)
        if ch + 2 < _NCHB:
            start(ch + 2)

    pltpu.sync_copy(obuf, out.at[:, pl.ds(base, _RPT)])


def _sc_call(tab, cellT, wf, bf):
    mesh = plsc.VectorSubcoreMesh(core_axis_name="c", subcore_axis_name="s")
    return pl.kernel(
        _sc_body,
        out_type=jax.ShapeDtypeStruct((3, _B), jnp.float32),
        mesh=mesh,
        compiler_params=pltpu.CompilerParams(use_tc_tiling_on_sc=False,
                                             needs_layout_passes=False),
        scratch_types=[
            pltpu.VMEM((2, _RPT), jnp.float32),
            pltpu.VMEM((_RPT,), jnp.float32),
            pltpu.VMEM((_RPT,), jnp.int32),
            pltpu.VMEM((_RPT,), jnp.int32),
            pltpu.VMEM((_CHB, _C), jnp.float32),
            pltpu.VMEM((_CHB, _C), jnp.float32),
            pltpu.VMEM((_CHB, _C), jnp.float32),
            pltpu.VMEM((_CHB, _C), jnp.float32),
            pltpu.VMEM((3 * _C,), jnp.float32),
            pltpu.VMEM((16,), jnp.float32),
            pltpu.VMEM((3, _RPT), jnp.float32),
            pltpu.SemaphoreType.DMA,
            pltpu.SemaphoreType.DMA,
            pltpu.SemaphoreType.DMA,
            pltpu.SemaphoreType.DMA,
        ],
    )(tab, cellT, wf, bf)


def kernel(h, cell, W, b):
    Bn, C_h, Kn = h.shape
    tab = jnp.transpose(h, (0, 2, 1)).reshape(Bn * Kn, C_h)
    cellT = cell.T
    wf = W.reshape(3 * C_h)
    bf = jnp.zeros((16,), jnp.float32).at[:3].set(b)
    o3 = _sc_call(tab, cellT, wf, bf)
    return o3.T
